# Initial kernel scaffold; baseline (speedup 1.0000x reference)
import jax
import jax.numpy as jnp
from jax.experimental import pallas as pl


def kernel(x, lengths, embed_table, W, b):
    raise NotImplementedError("write your pallas kernel here")


# speedup vs baseline: 1.3426x; 1.3426x over previous
import functools

import jax
import jax.numpy as jnp
from jax import lax
from jax.experimental import pallas as pl
from jax.experimental.pallas import tpu as pltpu
from jax.experimental.pallas import tpu_sc as plsc

_VOCAB = 1000000
_EMBED = 64
_SEQ = 200
_BATCH = 4096

_ROWS3 = _VOCAB // _EMBED
_RB = 125
_TC_GRID = _ROWS3 // _RB


def _tc_matvec_body(e_ref, w_ref, o_ref):
    w = w_ref[0]
    o_ref[...] = jnp.sum(e_ref[...] * w[None, None, :], axis=-1)[None]


def _tc_matvec(embed_table, W):
    e3 = embed_table.reshape(_ROWS3, _EMBED, _EMBED)
    p2 = pl.pallas_call(
        _tc_matvec_body,
        grid=(_TC_GRID,),
        in_specs=[
            pl.BlockSpec((_RB, _EMBED, _EMBED), lambda i: (i, 0, 0)),
            pl.BlockSpec((1, _EMBED), lambda i: (0, 0)),
        ],
        out_specs=pl.BlockSpec((1, _RB, _EMBED), lambda i: (i, 0, 0)),
        out_shape=jax.ShapeDtypeStruct((_TC_GRID, _RB, _EMBED), jnp.float32),
    )(e3, W)
    return p2.reshape(-1)


_NC = 2
_NS = 16
_NW = _NC * _NS
_CB = _BATCH // _NW


def _sc_pool(x, p, lengths, b16):
    mesh = plsc.VectorSubcoreMesh(core_axis_name="c", subcore_axis_name="s")

    @functools.partial(
        pl.kernel,
        out_type=jax.ShapeDtypeStruct((_BATCH,), jnp.float32),
        mesh=mesh,
        scratch_types=[
            pltpu.VMEM((_SEQ, _CB), jnp.int32),
            pltpu.VMEM((_SEQ, _CB), jnp.float32),
            pltpu.VMEM((_CB,), jnp.int32),
            pltpu.VMEM((16,), jnp.float32),
            pltpu.VMEM((_CB,), jnp.float32),
            pltpu.SemaphoreType.DMA,
        ],
    )
    def k(x_hbm, p_hbm, len_hbm, b_hbm, out_hbm, xv, gv, lenv, bv, outv, sem):
        wid = lax.axis_index("s") * _NC + lax.axis_index("c")
        base = wid * _CB
        pltpu.sync_copy(x_hbm.at[:, pl.ds(base, _CB)], xv)
        pltpu.sync_copy(len_hbm.at[pl.ds(base, _CB)], lenv)
        pltpu.sync_copy(b_hbm, bv)

        def fire(l, carry):
            pltpu.async_copy(p_hbm.at[xv.at[l]], gv.at[l], sem)
            return carry

        lax.fori_loop(0, _SEQ, fire, 0)

        def drain(l, carry):
            pltpu.make_async_copy(p_hbm.at[xv.at[0]], gv.at[0], sem).wait()
            return carry

        lax.fori_loop(0, _SEQ, drain, 0)
        bias = bv[...]
        for j in range(_CB // 16):
            sl = pl.ds(j * 16, 16)

            def body(l, acc):
                return acc + gv[l, sl]

            acc = lax.fori_loop(0, _SEQ, body, jnp.zeros((16,), jnp.float32))
            lf = lenv[sl].astype(jnp.float32)
            z = acc / lf + bias
            outv[sl] = 1.0 / (1.0 + jnp.exp(-z))
        pltpu.sync_copy(outv, out_hbm.at[pl.ds(base, _CB)])

    return k(x, p, lengths, b16)


def kernel(x, lengths, embed_table, W, b):
    p = _tc_matvec(embed_table, W.astype(jnp.float32))
    b16 = jnp.broadcast_to(b.astype(jnp.float32), (16,))
    return _sc_pool(x.astype(jnp.int32), p, lengths.astype(jnp.int32), b16)


# speedup vs baseline: 3.5390x; 2.6359x over previous
import functools

import jax
import jax.numpy as jnp
from jax import lax
from jax.experimental import pallas as pl
from jax.experimental.pallas import tpu as pltpu
from jax.experimental.pallas import tpu_sc as plsc

_VOCAB = 1000000
_EMBED = 64
_SEQ = 200
_BATCH = 4096

_VCOLS = 8192
_TCG = (_VOCAB + _VCOLS - 1) // _VCOLS


def _tc_matvec_body(e_ref, w_ref, o_ref):
    o_ref[...] = jnp.sum(e_ref[...] * w_ref[...], axis=0)


def _tc_matvec(embed_table, W):
    et = embed_table.T
    wt = W.reshape(_EMBED, 1)
    return pl.pallas_call(
        _tc_matvec_body,
        grid=(_TCG,),
        in_specs=[
            pl.BlockSpec((_EMBED, _VCOLS), lambda i: (0, i)),
            pl.BlockSpec((_EMBED, 1), lambda i: (0, 0)),
        ],
        out_specs=pl.BlockSpec((_VCOLS,), lambda i: (i,)),
        out_shape=jax.ShapeDtypeStruct((_VOCAB,), jnp.float32),
    )(et, wt)


_NC = 2
_NS = 16
_NW = _NC * _NS
_CB = _BATCH // _NW


def _sc_pool(x, p, lengths, b16):
    mesh = plsc.VectorSubcoreMesh(core_axis_name="c", subcore_axis_name="s")

    @functools.partial(
        pl.kernel,
        out_type=jax.ShapeDtypeStruct((_BATCH,), jnp.float32),
        mesh=mesh,
        scratch_types=[
            pltpu.VMEM((_SEQ, _CB), jnp.int32),
            pltpu.VMEM((_SEQ, _CB), jnp.float32),
            pltpu.VMEM((_CB,), jnp.int32),
            pltpu.VMEM((16,), jnp.float32),
            pltpu.VMEM((_CB,), jnp.float32),
            pltpu.SemaphoreType.DMA,
        ],
    )
    def k(x_hbm, p_hbm, len_hbm, b_hbm, out_hbm, xv, gv, lenv, bv, outv, sem):
        wid = lax.axis_index("s") * _NC + lax.axis_index("c")
        base = wid * _CB
        pltpu.sync_copy(x_hbm.at[:, pl.ds(base, _CB)], xv)
        pltpu.sync_copy(len_hbm.at[pl.ds(base, _CB)], lenv)
        pltpu.sync_copy(b_hbm, bv)

        def fire(l, carry):
            pltpu.async_copy(p_hbm.at[xv.at[l]], gv.at[l], sem)
            return carry

        lax.fori_loop(0, _SEQ, fire, 0)

        def drain(l, carry):
            pltpu.make_async_copy(p_hbm.at[xv.at[0]], gv.at[0], sem).wait()
            return carry

        lax.fori_loop(0, _SEQ, drain, 0)
        bias = bv[...]
        for j in range(_CB // 16):
            sl = pl.ds(j * 16, 16)

            def body(l, acc):
                return acc + gv[l, sl]

            acc = lax.fori_loop(0, _SEQ, body, jnp.zeros((16,), jnp.float32))
            lf = lenv[sl].astype(jnp.float32)
            z = acc / lf + bias
            outv[sl] = 1.0 / (1.0 + jnp.exp(-z))
        pltpu.sync_copy(outv, out_hbm.at[pl.ds(base, _CB)])

    return k(x, p, lengths, b16)


def kernel(x, lengths, embed_table, W, b):
    p = _tc_matvec(embed_table, W.astype(jnp.float32))
    b16 = jnp.broadcast_to(b.astype(jnp.float32), (16,))
    return _sc_pool(x.astype(jnp.int32), p, lengths.astype(jnp.int32), b16)


# speedup vs baseline: 4.7358x; 1.3382x over previous
import functools

import jax
import jax.numpy as jnp
from jax import lax
from jax.experimental import pallas as pl
from jax.experimental.pallas import tpu as pltpu
from jax.experimental.pallas import tpu_sc as plsc

_VOCAB = 1000000
_EMBED = 64
_SEQ = 200
_BATCH = 4096

_VCOLS = 32768
_TCG = (_VOCAB + _VCOLS - 1) // _VCOLS


def _tc_matvec_body(e_ref, w_ref, o_ref):
    o_ref[...] = jnp.sum(e_ref[...] * w_ref[...], axis=0)


def _tc_matvec(embed_table, W):
    et = embed_table.T
    wt = W.reshape(_EMBED, 1)
    return pl.pallas_call(
        _tc_matvec_body,
        grid=(_TCG,),
        in_specs=[
            pl.BlockSpec((_EMBED, _VCOLS), lambda i: (0, i)),
            pl.BlockSpec((_EMBED, 1), lambda i: (0, 0)),
        ],
        out_specs=pl.BlockSpec((_VCOLS,), lambda i: (i,)),
        out_shape=jax.ShapeDtypeStruct((_VOCAB,), jnp.float32),
    )(et, wt)


_NC = 2
_NS = 16
_NW = _NC * _NS
_CB = _BATCH // _NW


def _sc_pool(x, p, lengths, b16):
    mesh = plsc.VectorSubcoreMesh(core_axis_name="c", subcore_axis_name="s")

    @functools.partial(
        pl.kernel,
        out_type=jax.ShapeDtypeStruct((_BATCH,), jnp.float32),
        mesh=mesh,
        scratch_types=[
            pltpu.VMEM((_SEQ, _CB), jnp.int32),
            pltpu.VMEM((_SEQ, _CB), jnp.float32),
            pltpu.VMEM((_CB,), jnp.int32),
            pltpu.VMEM((16,), jnp.float32),
            pltpu.VMEM((_CB,), jnp.float32),
            pltpu.SemaphoreType.DMA,
        ],
    )
    def k(x_hbm, p_hbm, len_hbm, b_hbm, out_hbm, xv, gv, lenv, bv, outv, sem):
        wid = lax.axis_index("s") * _NC + lax.axis_index("c")
        base = wid * _CB
        pltpu.sync_copy(x_hbm.at[:, pl.ds(base, _CB)], xv)
        pltpu.sync_copy(len_hbm.at[pl.ds(base, _CB)], lenv)
        pltpu.sync_copy(b_hbm, bv)

        def fire(l, carry):
            pltpu.async_copy(p_hbm.at[xv.at[l]], gv.at[l], sem)
            return carry

        lax.fori_loop(0, _SEQ, fire, 0)

        def drain(l, carry):
            pltpu.make_async_copy(p_hbm.at[xv.at[0]], gv.at[0], sem).wait()
            return carry

        lax.fori_loop(0, _SEQ, drain, 0)
        bias = bv[...]
        for j in range(_CB // 16):
            sl = pl.ds(j * 16, 16)

            def body(l, acc):
                return acc + gv[l, sl]

            acc = lax.fori_loop(0, _SEQ, body, jnp.zeros((16,), jnp.float32))
            lf = lenv[sl].astype(jnp.float32)
            z = acc / lf + bias
            outv[sl] = 1.0 / (1.0 + jnp.exp(-z))
        pltpu.sync_copy(outv, out_hbm.at[pl.ds(base, _CB)])

    return k(x, p, lengths, b16)


def kernel(x, lengths, embed_table, W, b):
    p = _tc_matvec(embed_table, W.astype(jnp.float32))
    b16 = jnp.broadcast_to(b.astype(jnp.float32), (16,))
    return _sc_pool(x.astype(jnp.int32), p, lengths.astype(jnp.int32), b16)
